# baseline (speedup 1.0000x reference)
import functools

import jax
import jax.numpy as jnp
from jax import lax
from jax.experimental import pallas as pl
from jax.experimental.pallas import tpu as pltpu
from jax.experimental.pallas import tpu_sc as plsc

N = 10000
E = 320000
FEAT = 128
HID = 256
EMB = 128

NB = 1000
GRID = N // NB

NSC = 2
NSS = 16
CH = 128
NCH = 160
NCHT = NCH * NSS
EP = NCHT * CH
ACC_ROWS = 10240
ZPT = ACC_ROWS // NSS
SECN = 16
CH2 = 64
NBUF = 4
SROW = 16
CPW = NCHT // (NSC * NSS)
TRASH = N


def _sc_mesh():
    return plsc.VectorSubcoreMesh(core_axis_name="c", subcore_axis_name="s",
                                  num_cores=NSC, num_subcores=NSS)


def _make_agg(dh, split):
    nch = NCH if split else CPW
    nsec = nch // SROW
    nck = 2 * SROW
    @functools.partial(
        pl.kernel,
        out_type=jax.ShapeDtypeStruct((NSC, ACC_ROWS, dh), jnp.float32),
        mesh=_sc_mesh(),
        scratch_types=[
            pltpu.VMEM((SROW, 2, CH2), jnp.int32),
            pltpu.VMEM((SROW, 2, CH2), jnp.int32),
            [pltpu.VMEM((CH2, dh), jnp.float32)] * NBUF,
            [pltpu.SemaphoreType.DMA] * NBUF,
            [pltpu.SemaphoreType.DMA] * NBUF,
            pltpu.VMEM_SHARED((ACC_ROWS, dh), jnp.float32),
        ],
    )
    def agg(h, ridx, cidx, zeros, out, row_v, col_v, bufs, gsem, ssem, acc):
        c = lax.axis_index("c")
        s = lax.axis_index("s")
        base = s * nch if split else (c * NSS + s) * CPW
        pltpu.sync_copy(zeros.at[pl.ds(s * ZPT, ZPT)],
                        acc.at[pl.ds(s * ZPT, ZPT)])
        plsc.subcore_barrier()

        def section(t, carry):
            sl = pl.ds(base + t * SROW, SROW)
            if split:
                pltpu.sync_copy(ridx.at[c, sl], row_v)
            else:
                pltpu.sync_copy(ridx.at[sl], row_v)
            pltpu.sync_copy(cidx.at[sl], col_v)

            gd = [None] * nck
            sd = [None] * nck
            for m in range(nck + NBUF - 1):
                if m < nck:
                    slot = m % NBUF
                    if m >= NBUF:
                        sd[m - NBUF].wait()
                    gd[m] = pltpu.async_copy(
                        h.at[row_v.at[m // 2, m % 2]], bufs[slot], gsem[slot])
                mm = m - (NBUF - 1)
                if 0 <= mm < nck:
                    gd[mm].wait()
                    sd[mm] = pltpu.async_copy(
                        bufs[mm % NBUF], acc.at[col_v.at[mm // 2, mm % 2]],
                        ssem[mm % NBUF], add=True)
            for mm in range(nck - NBUF, nck):
                sd[mm].wait()
            return carry

        lax.fori_loop(0, nsec, section, 0)
        plsc.subcore_barrier()
        pltpu.sync_copy(acc.at[pl.ds(s * ZPT, ZPT)],
                        out.at[c, pl.ds(s * ZPT, ZPT)])

    return agg


def _make_deg():
    @functools.partial(
        pl.kernel,
        out_type=jax.ShapeDtypeStruct((NSC, ACC_ROWS, 128), jnp.float32),
        mesh=_sc_mesh(),
        scratch_types=[
            pltpu.VMEM((SECN, CH), jnp.int32),
            pltpu.VMEM((CH, 128), jnp.float32),
            pltpu.VMEM_SHARED((ACC_ROWS, 128), jnp.float32),
            pltpu.SemaphoreType.DMA,
        ],
    )
    def deg(cidx, ones, zeros, out, col_v, ones_v, acc, sem):
        c = lax.axis_index("c")
        s = lax.axis_index("s")
        base = (c * NSS + s) * CPW
        pltpu.sync_copy(ones, ones_v)
        pltpu.sync_copy(zeros.at[pl.ds(s * ZPT, ZPT)],
                        acc.at[pl.ds(s * ZPT, ZPT)])
        plsc.subcore_barrier()

        def section(t, carry):
            pltpu.sync_copy(cidx.at[pl.ds(base + t * SECN, SECN)], col_v)
            descs = [pltpu.async_copy(ones_v, acc.at[col_v.at[j]], sem,
                                      add=True) for j in range(SECN)]
            for d in descs:
                d.wait()
            return carry

        lax.fori_loop(0, CPW // SECN, section, 0)
        plsc.subcore_barrier()
        pltpu.sync_copy(acc.at[pl.ds(s * ZPT, ZPT)],
                        out.at[c, pl.ds(s * ZPT, ZPT)])

    return deg


_AGG128 = _make_agg(HID // 2, split=True)
_AGGF = _make_agg(EMB, split=False)
_DEG = _make_deg()


def _tc_call(body, out_shape, in_specs, out_specs):
    return pl.pallas_call(
        body,
        out_shape=out_shape,
        grid=(GRID,),
        in_specs=in_specs,
        out_specs=out_specs,
    )


def _full(shape):
    return pl.BlockSpec(shape, lambda j: tuple(0 for _ in shape))


def _k0_body(deg_ref, dis_ref):
    dis_ref[...] = lax.rsqrt(deg_ref[0, :, :1] + deg_ref[1, :, :1] + 1.0)


def _k1_body(x_ref, w_ref, dis_ref, out_ref):
    h = jnp.dot(x_ref[...], w_ref[...], preferred_element_type=jnp.float32)
    h = h * dis_ref[...]
    half = h.shape[1] // 2
    out_ref[0] = h[:, :half]
    out_ref[1] = h[:, half:]


def _k2_body(agg_ref, h_ref, dis_ref, b_ref, w_ref, out_ref):
    t = jnp.concatenate([agg_ref[0] + h_ref[0], agg_ref[1] + h_ref[1]], axis=1)
    act = jnp.maximum(t * dis_ref[...] + b_ref[...], 0.0)
    h = jnp.dot(act, w_ref[...], preferred_element_type=jnp.float32)
    h = h * dis_ref[...]
    half = h.shape[1] // 2
    out_ref[0] = h[:, :half]
    out_ref[1] = h[:, half:]


def _k3_body(agg_ref, h_ref, dis_ref, b_ref, w_ref, out_ref):
    t = jnp.concatenate([agg_ref[0] + h_ref[0], agg_ref[1] + h_ref[1]], axis=1)
    act = jnp.maximum(t * dis_ref[...] + b_ref[...], 0.0)
    h = jnp.dot(act, w_ref[...], preferred_element_type=jnp.float32)
    out_ref[...] = h * dis_ref[...]


def _k4_body(agg_ref, h_ref, dis_ref, b_ref, fw1_ref, fb1_ref, fw2_ref,
             fb2_ref, out_ref):
    t = agg_ref[0] + agg_ref[1] + h_ref[...]
    act = jnp.maximum(t * dis_ref[...] + b_ref[...], 0.0)
    hf = jnp.dot(act, fw1_ref[...], preferred_element_type=jnp.float32)
    hf = jnp.maximum(hf + fb1_ref[...], 0.0)
    out = jnp.dot(hf, fw2_ref[...], preferred_element_type=jnp.float32)
    out_ref[...] = out + fb2_ref[...]


def _dis_kernel(deg):
    return pl.pallas_call(
        _k0_body,
        out_shape=jax.ShapeDtypeStruct((N, 1), jnp.float32),
        grid=(GRID,),
        in_specs=[pl.BlockSpec((2, NB, 128), lambda j: (0, j, 0))],
        out_specs=pl.BlockSpec((NB, 1), lambda j: (j, 0)),
    )(deg)


def _h1_kernel(x, w, dis):
    din, dout = w.shape
    return _tc_call(
        _k1_body,
        jax.ShapeDtypeStruct((2, N, dout // 2), jnp.float32),
        [
            pl.BlockSpec((NB, din), lambda j: (j, 0)),
            _full((din, dout)),
            pl.BlockSpec((NB, 1), lambda j: (j, 0)),
        ],
        pl.BlockSpec((2, NB, dout // 2), lambda j: (0, j, 0)),
    )(x, w, dis)


def _mid_kernel(agg, h, dis, b, w):
    din, dout = w.shape
    return _tc_call(
        _k2_body,
        jax.ShapeDtypeStruct((2, N, dout // 2), jnp.float32),
        [
            pl.BlockSpec((2, NB, din // 2), lambda j: (0, j, 0)),
            pl.BlockSpec((2, NB, din // 2), lambda j: (0, j, 0)),
            pl.BlockSpec((NB, 1), lambda j: (j, 0)),
            _full((1, din)),
            _full((din, dout)),
        ],
        pl.BlockSpec((2, NB, dout // 2), lambda j: (0, j, 0)),
    )(agg, h, dis, b, w)


def _mid_full_kernel(agg, h, dis, b, w):
    din, dout = w.shape
    return _tc_call(
        _k3_body,
        jax.ShapeDtypeStruct((N, dout), jnp.float32),
        [
            pl.BlockSpec((2, NB, din // 2), lambda j: (0, j, 0)),
            pl.BlockSpec((2, NB, din // 2), lambda j: (0, j, 0)),
            pl.BlockSpec((NB, 1), lambda j: (j, 0)),
            _full((1, din)),
            _full((din, dout)),
        ],
        pl.BlockSpec((NB, dout), lambda j: (j, 0)),
    )(agg, h, dis, b, w)


def _head_kernel(agg, h, dis, b3, fw1, fb1, fw2, fb2):
    return _tc_call(
        _k4_body,
        jax.ShapeDtypeStruct((N, EMB), jnp.float32),
        [
            pl.BlockSpec((2, NB, EMB), lambda j: (0, j, 0)),
            pl.BlockSpec((NB, EMB), lambda j: (j, 0)),
            pl.BlockSpec((NB, 1), lambda j: (j, 0)),
            _full((1, EMB)),
            _full((EMB, HID)),
            _full((1, HID)),
            _full((HID, EMB)),
            _full((1, EMB)),
        ],
        pl.BlockSpec((NB, EMB), lambda j: (j, 0)),
    )(agg, h, dis, b3, fw1, fb1, fw2, fb2)


def kernel(x, edge_index, W1, b1, W2, b2, W3, b3, fcW1, fcb1, fcW2, fcb2):
    row = edge_index[0]
    col = edge_index[1]

    pad = EP - E
    rowp = jnp.concatenate([row, jnp.zeros((pad,), row.dtype)])
    colp = jnp.concatenate([col, jnp.full((pad,), TRASH, col.dtype)])
    ridx1 = rowp.reshape(NCHT, 2, CH2)
    ridx2 = jnp.stack([rowp, rowp + N]).reshape(NSC, NCHT, 2, CH2)
    cidx = colp.reshape(NCHT, 2, CH2)
    cidx128 = colp.reshape(NCHT, CH)
    ones128 = jnp.ones((CH, 128), jnp.float32)
    zeros128 = jnp.zeros((ACC_ROWS, 128), jnp.float32)

    deg = _DEG(cidx128, ones128, zeros128)
    dis = _dis_kernel(deg)

    h1 = _h1_kernel(x, W1, dis)
    agg1 = _AGG128(h1.reshape(2 * N, HID // 2), ridx2, cidx, zeros128)
    h2 = _mid_kernel(agg1, h1, dis, b1.reshape(1, HID), W2)
    agg2 = _AGG128(h2.reshape(2 * N, HID // 2), ridx2, cidx, zeros128)
    h3 = _mid_full_kernel(agg2, h2, dis, b2.reshape(1, HID), W3)
    agg3 = _AGGF(h3, ridx1, cidx, zeros128)
    out = _head_kernel(agg3, h3, dis, b3.reshape(1, EMB),
                       fcW1, fcb1.reshape(1, HID), fcW2, fcb2.reshape(1, EMB))
    return out


# speedup vs baseline: 1.0252x; 1.0252x over previous
import functools

import jax
import jax.numpy as jnp
from jax import lax
from jax.experimental import pallas as pl
from jax.experimental.pallas import tpu as pltpu
from jax.experimental.pallas import tpu_sc as plsc

N = 10000
E = 320000
FEAT = 128
HID = 256
EMB = 128

NB = 1000
GRID = N // NB

NSC = 2
NSS = 16
CH = 128
NCH = 160
NCHT = NCH * NSS
EP = NCHT * CH
ACC_ROWS = 10240
ZPT = ACC_ROWS // NSS
SECN = 16
CH2 = 64
NBUF = 4
SROW = 16
CPW = NCHT // (NSC * NSS)
TRASH = N


def _sc_mesh():
    return plsc.VectorSubcoreMesh(core_axis_name="c", subcore_axis_name="s",
                                  num_cores=NSC, num_subcores=NSS)


def _make_agg(dh, split):
    nch = NCH if split else CPW
    secn = SECN if split else SECN // 2
    nsec = nch // secn
    @functools.partial(
        pl.kernel,
        out_type=jax.ShapeDtypeStruct((NSC, ACC_ROWS, dh), jnp.float32),
        mesh=_sc_mesh(),
        scratch_types=[
            [pltpu.VMEM((secn, CH), jnp.int32)] * 2,
            [pltpu.VMEM((secn, CH), jnp.int32)] * 2,
            [pltpu.VMEM((CH, dh), jnp.float32)] * 2,
            [pltpu.SemaphoreType.DMA] * 2,
            [pltpu.SemaphoreType.DMA] * 2,
            [pltpu.SemaphoreType.DMA] * 2,
            pltpu.VMEM_SHARED((ACC_ROWS, dh), jnp.float32),
        ],
    )
    def agg(h, ridx, cidx, zeros, out, rows, cols, bufs, gsem, ssem, isem,
            acc):
        c = lax.axis_index("c")
        s = lax.axis_index("s")
        base = s * nch if split else (c * NSS + s) * CPW

        def ridx_src(t):
            sl = pl.ds(base + t * secn, secn)
            return ridx.at[c, sl] if split else ridx.at[sl]

        def issue_idx(t, par):
            pltpu.async_copy(ridx_src(t), rows[par], isem[par])
            pltpu.async_copy(cidx.at[pl.ds(base + t * secn, secn)], cols[par],
                             isem[par])

        def wait_idx(par):
            pltpu.make_async_copy(ridx_src(0), rows[par], isem[par]).wait()
            pltpu.make_async_copy(ridx_src(0), cols[par], isem[par]).wait()

        def section(t, par):
            wait_idx(par)

            @pl.when(t + 1 < nsec)
            def _():
                issue_idx(t + 1, 1 - par)

            row_v = rows[par]
            col_v = cols[par]
            g = [None, None]
            sc = [None, None]
            g[0] = pltpu.async_copy(h.at[row_v.at[0]], bufs[0], gsem[0])
            for j in range(secn):
                p = j % 2
                q = (j + 1) % 2
                if j + 1 < secn:
                    if j >= 1:
                        sc[q].wait()
                    g[q] = pltpu.async_copy(h.at[row_v.at[j + 1]], bufs[q],
                                            gsem[q])
                g[p].wait()
                sc[p] = pltpu.async_copy(bufs[p], acc.at[col_v.at[j]],
                                         ssem[p], add=True)
            sc[(secn - 2) % 2].wait()
            sc[(secn - 1) % 2].wait()

        issue_idx(0, 0)
        pltpu.sync_copy(zeros.at[pl.ds(s * ZPT, ZPT)],
                        acc.at[pl.ds(s * ZPT, ZPT)])
        plsc.subcore_barrier()

        def pair(u, carry):
            section(2 * u, 0)
            section(2 * u + 1, 1)
            return carry

        lax.fori_loop(0, nsec // 2, pair, 0)
        plsc.subcore_barrier()
        pltpu.sync_copy(acc.at[pl.ds(s * ZPT, ZPT)],
                        out.at[c, pl.ds(s * ZPT, ZPT)])

    return agg


def _make_deg():
    @functools.partial(
        pl.kernel,
        out_type=jax.ShapeDtypeStruct((NSC, ACC_ROWS, 128), jnp.float32),
        mesh=_sc_mesh(),
        scratch_types=[
            pltpu.VMEM((SECN, CH), jnp.int32),
            pltpu.VMEM((CH, 128), jnp.float32),
            pltpu.VMEM_SHARED((ACC_ROWS, 128), jnp.float32),
            pltpu.SemaphoreType.DMA,
        ],
    )
    def deg(cidx, ones, zeros, out, col_v, ones_v, acc, sem):
        c = lax.axis_index("c")
        s = lax.axis_index("s")
        base = (c * NSS + s) * CPW
        pltpu.sync_copy(ones, ones_v)
        pltpu.sync_copy(zeros.at[pl.ds(s * ZPT, ZPT)],
                        acc.at[pl.ds(s * ZPT, ZPT)])
        plsc.subcore_barrier()

        def section(t, carry):
            pltpu.sync_copy(cidx.at[pl.ds(base + t * SECN, SECN)], col_v)
            descs = [pltpu.async_copy(ones_v, acc.at[col_v.at[j]], sem,
                                      add=True) for j in range(SECN)]
            for d in descs:
                d.wait()
            return carry

        lax.fori_loop(0, CPW // SECN, section, 0)
        plsc.subcore_barrier()
        pltpu.sync_copy(acc.at[pl.ds(s * ZPT, ZPT)],
                        out.at[c, pl.ds(s * ZPT, ZPT)])

    return deg


_AGG128 = _make_agg(HID // 2, split=True)
_AGGF = _make_agg(EMB, split=False)
_DEG = _make_deg()


def _tc_call(body, out_shape, in_specs, out_specs):
    return pl.pallas_call(
        body,
        out_shape=out_shape,
        grid=(GRID,),
        in_specs=in_specs,
        out_specs=out_specs,
    )


def _full(shape):
    return pl.BlockSpec(shape, lambda j: tuple(0 for _ in shape))


def _k0_body(deg_ref, dis_ref):
    dis_ref[...] = lax.rsqrt(deg_ref[0, :, :1] + deg_ref[1, :, :1] + 1.0)


def _k1_body(x_ref, w_ref, dis_ref, out_ref):
    h = jnp.dot(x_ref[...], w_ref[...], preferred_element_type=jnp.float32)
    h = h * dis_ref[...]
    half = h.shape[1] // 2
    out_ref[0] = h[:, :half]
    out_ref[1] = h[:, half:]


def _k2_body(agg_ref, h_ref, dis_ref, b_ref, w_ref, out_ref):
    t = jnp.concatenate([agg_ref[0] + h_ref[0], agg_ref[1] + h_ref[1]], axis=1)
    act = jnp.maximum(t * dis_ref[...] + b_ref[...], 0.0)
    h = jnp.dot(act, w_ref[...], preferred_element_type=jnp.float32)
    h = h * dis_ref[...]
    half = h.shape[1] // 2
    out_ref[0] = h[:, :half]
    out_ref[1] = h[:, half:]


def _k3_body(agg_ref, h_ref, dis_ref, b_ref, w_ref, out_ref):
    t = jnp.concatenate([agg_ref[0] + h_ref[0], agg_ref[1] + h_ref[1]], axis=1)
    act = jnp.maximum(t * dis_ref[...] + b_ref[...], 0.0)
    h = jnp.dot(act, w_ref[...], preferred_element_type=jnp.float32)
    out_ref[...] = h * dis_ref[...]


def _k4_body(agg_ref, h_ref, dis_ref, b_ref, fw1_ref, fb1_ref, fw2_ref,
             fb2_ref, out_ref):
    t = agg_ref[0] + agg_ref[1] + h_ref[...]
    act = jnp.maximum(t * dis_ref[...] + b_ref[...], 0.0)
    hf = jnp.dot(act, fw1_ref[...], preferred_element_type=jnp.float32)
    hf = jnp.maximum(hf + fb1_ref[...], 0.0)
    out = jnp.dot(hf, fw2_ref[...], preferred_element_type=jnp.float32)
    out_ref[...] = out + fb2_ref[...]


def _dis_kernel(deg):
    return pl.pallas_call(
        _k0_body,
        out_shape=jax.ShapeDtypeStruct((N, 1), jnp.float32),
        grid=(GRID,),
        in_specs=[pl.BlockSpec((2, NB, 128), lambda j: (0, j, 0))],
        out_specs=pl.BlockSpec((NB, 1), lambda j: (j, 0)),
    )(deg)


def _h1_kernel(x, w, dis):
    din, dout = w.shape
    return _tc_call(
        _k1_body,
        jax.ShapeDtypeStruct((2, N, dout // 2), jnp.float32),
        [
            pl.BlockSpec((NB, din), lambda j: (j, 0)),
            _full((din, dout)),
            pl.BlockSpec((NB, 1), lambda j: (j, 0)),
        ],
        pl.BlockSpec((2, NB, dout // 2), lambda j: (0, j, 0)),
    )(x, w, dis)


def _mid_kernel(agg, h, dis, b, w):
    din, dout = w.shape
    return _tc_call(
        _k2_body,
        jax.ShapeDtypeStruct((2, N, dout // 2), jnp.float32),
        [
            pl.BlockSpec((2, NB, din // 2), lambda j: (0, j, 0)),
            pl.BlockSpec((2, NB, din // 2), lambda j: (0, j, 0)),
            pl.BlockSpec((NB, 1), lambda j: (j, 0)),
            _full((1, din)),
            _full((din, dout)),
        ],
        pl.BlockSpec((2, NB, dout // 2), lambda j: (0, j, 0)),
    )(agg, h, dis, b, w)


def _mid_full_kernel(agg, h, dis, b, w):
    din, dout = w.shape
    return _tc_call(
        _k3_body,
        jax.ShapeDtypeStruct((N, dout), jnp.float32),
        [
            pl.BlockSpec((2, NB, din // 2), lambda j: (0, j, 0)),
            pl.BlockSpec((2, NB, din // 2), lambda j: (0, j, 0)),
            pl.BlockSpec((NB, 1), lambda j: (j, 0)),
            _full((1, din)),
            _full((din, dout)),
        ],
        pl.BlockSpec((NB, dout), lambda j: (j, 0)),
    )(agg, h, dis, b, w)


def _head_kernel(agg, h, dis, b3, fw1, fb1, fw2, fb2):
    return _tc_call(
        _k4_body,
        jax.ShapeDtypeStruct((N, EMB), jnp.float32),
        [
            pl.BlockSpec((2, NB, EMB), lambda j: (0, j, 0)),
            pl.BlockSpec((NB, EMB), lambda j: (j, 0)),
            pl.BlockSpec((NB, 1), lambda j: (j, 0)),
            _full((1, EMB)),
            _full((EMB, HID)),
            _full((1, HID)),
            _full((HID, EMB)),
            _full((1, EMB)),
        ],
        pl.BlockSpec((NB, EMB), lambda j: (j, 0)),
    )(agg, h, dis, b3, fw1, fb1, fw2, fb2)


def kernel(x, edge_index, W1, b1, W2, b2, W3, b3, fcW1, fcb1, fcW2, fcb2):
    row = edge_index[0]
    col = edge_index[1]

    pad = EP - E
    rowp = jnp.concatenate([row, jnp.zeros((pad,), row.dtype)])
    colp = jnp.concatenate([col, jnp.full((pad,), TRASH, col.dtype)])
    ridx1 = rowp.reshape(NCHT, CH)
    ridx2 = jnp.stack([rowp, rowp + N]).reshape(NSC, NCHT, CH)
    cidx = colp.reshape(NCHT, CH)
    ones128 = jnp.ones((CH, 128), jnp.float32)
    zeros128 = jnp.zeros((ACC_ROWS, 128), jnp.float32)

    deg = _DEG(cidx, ones128, zeros128)
    dis = _dis_kernel(deg)

    h1 = _h1_kernel(x, W1, dis)
    agg1 = _AGG128(h1.reshape(2 * N, HID // 2), ridx2, cidx, zeros128)
    h2 = _mid_kernel(agg1, h1, dis, b1.reshape(1, HID), W2)
    agg2 = _AGG128(h2.reshape(2 * N, HID // 2), ridx2, cidx, zeros128)
    h3 = _mid_full_kernel(agg2, h2, dis, b2.reshape(1, HID), W3)
    agg3 = _AGGF(h3, ridx1, cidx, zeros128)
    out = _head_kernel(agg3, h3, dis, b3.reshape(1, EMB),
                       fcW1, fcb1.reshape(1, HID), fcW2, fcb2.reshape(1, EMB))
    return out


# speedup vs baseline: 2.5816x; 2.5181x over previous
import functools

import jax
import jax.numpy as jnp
from jax import lax
from jax.experimental import pallas as pl
from jax.experimental.pallas import tpu as pltpu
from jax.experimental.pallas import tpu_sc as plsc

N = 10000
E = 320000
FEAT = 128
HID = 256
EMB = 128

NB = 1000
GRID = N // NB

NSC = 2
NSS = 16
CH = 128
NCH = 160
NCHT = NCH * NSS
EP = NCHT * CH
ACC_ROWS = 10240
ZPT = ACC_ROWS // NSS
SECN = 16
CH2 = 64
NBUF = 4
SROW = 16
CPW = NCHT // (NSC * NSS)
TRASH = N


def _sc_mesh():
    return plsc.VectorSubcoreMesh(core_axis_name="c", subcore_axis_name="s",
                                  num_cores=NSC, num_subcores=NSS)


def _make_agg(dh, split):
    nch = NCH if split else CPW
    secn = SECN if split else SECN // 2
    nsec = nch // secn
    @functools.partial(
        pl.kernel,
        out_type=jax.ShapeDtypeStruct((NSC, ACC_ROWS, dh), jnp.float32),
        mesh=_sc_mesh(),
        scratch_types=[
            [pltpu.VMEM((secn, CH), jnp.int32)] * 2,
            [pltpu.VMEM((secn, CH), jnp.int32)] * 2,
            [pltpu.VMEM((CH, dh), jnp.float32)] * 2,
            [pltpu.SemaphoreType.DMA] * 2,
            [pltpu.SemaphoreType.DMA] * 2,
            [pltpu.SemaphoreType.DMA] * 2,
            pltpu.VMEM_SHARED((ACC_ROWS, dh), jnp.float32),
        ],
    )
    def agg(h, ridx, cidx, zeros, out, rows, cols, bufs, gsem, ssem, isem,
            acc):
        c = lax.axis_index("c")
        s = lax.axis_index("s")
        base = s * nch if split else (c * NSS + s) * CPW

        def ridx_src(t):
            sl = pl.ds(base + t * secn, secn)
            return ridx.at[c, sl] if split else ridx.at[sl]

        def issue_idx(t, par):
            pltpu.async_copy(ridx_src(t), rows[par], isem[par])
            pltpu.async_copy(cidx.at[pl.ds(base + t * secn, secn)], cols[par],
                             isem[par])

        def wait_idx(par):
            pltpu.make_async_copy(ridx_src(0), rows[par], isem[par]).wait()
            pltpu.make_async_copy(ridx_src(0), cols[par], isem[par]).wait()

        def section(t, par):
            wait_idx(par)

            @pl.when(t + 1 < nsec)
            def _():
                issue_idx(t + 1, 1 - par)

            row_v = rows[par]
            col_v = cols[par]
            g = [None, None]
            sc = [None, None]
            g[0] = pltpu.async_copy(h.at[row_v.at[0]], bufs[0], gsem[0])
            for j in range(secn):
                p = j % 2
                q = (j + 1) % 2
                if j + 1 < secn:
                    if j >= 1:
                        sc[q].wait()
                    g[q] = pltpu.async_copy(h.at[row_v.at[j + 1]], bufs[q],
                                            gsem[q])
                g[p].wait()
                sc[p] = pltpu.async_copy(bufs[p], acc.at[col_v.at[j]],
                                         ssem[p], add=True)
            sc[(secn - 2) % 2].wait()
            sc[(secn - 1) % 2].wait()

        issue_idx(0, 0)
        pltpu.sync_copy(zeros.at[pl.ds(s * ZPT, ZPT)],
                        acc.at[pl.ds(s * ZPT, ZPT)])
        plsc.subcore_barrier()

        def pair(u, carry):
            section(2 * u, 0)
            section(2 * u + 1, 1)
            return carry

        lax.fori_loop(0, nsec // 2, pair, 0)
        plsc.subcore_barrier()
        pltpu.sync_copy(acc.at[pl.ds(s * ZPT, ZPT)],
                        out.at[c, pl.ds(s * ZPT, ZPT)])

    return agg


def _make_deg():
    @functools.partial(
        pl.kernel,
        out_type=jax.ShapeDtypeStruct((NSC, ACC_ROWS, 128), jnp.float32),
        mesh=_sc_mesh(),
        scratch_types=[
            pltpu.VMEM((SECN, CH), jnp.int32),
            pltpu.VMEM((CH, 128), jnp.float32),
            pltpu.VMEM_SHARED((ACC_ROWS, 128), jnp.float32),
            pltpu.SemaphoreType.DMA,
        ],
    )
    def deg(cidx, ones, zeros, out, col_v, ones_v, acc, sem):
        c = lax.axis_index("c")
        s = lax.axis_index("s")
        base = (c * NSS + s) * CPW
        pltpu.sync_copy(ones, ones_v)
        pltpu.sync_copy(zeros.at[pl.ds(s * ZPT, ZPT)],
                        acc.at[pl.ds(s * ZPT, ZPT)])
        plsc.subcore_barrier()

        def section(t, carry):
            pltpu.sync_copy(cidx.at[pl.ds(base + t * SECN, SECN)], col_v)
            descs = [pltpu.async_copy(ones_v, acc.at[col_v.at[j]], sem,
                                      add=True) for j in range(SECN)]
            for d in descs:
                d.wait()
            return carry

        lax.fori_loop(0, CPW // SECN, section, 0)
        plsc.subcore_barrier()
        pltpu.sync_copy(acc.at[pl.ds(s * ZPT, ZPT)],
                        out.at[c, pl.ds(s * ZPT, ZPT)])

    return deg


_AGG128 = _make_agg(HID // 2, split=True)
_AGGF = _make_agg(EMB, split=False)
_DEG = _make_deg()


def _tc_call(body, out_shape, in_specs, out_specs):
    return pl.pallas_call(
        body,
        out_shape=out_shape,
        grid=(GRID,),
        in_specs=in_specs,
        out_specs=out_specs,
    )


def _full(shape):
    return pl.BlockSpec(shape, lambda j: tuple(0 for _ in shape))


def _k0_body(deg_ref, dis_ref):
    dis_ref[...] = lax.rsqrt(deg_ref[0, :, :1] + deg_ref[1, :, :1] + 1.0)


def _k1_body(x_ref, w_ref, dis_ref, out_ref):
    h = jnp.dot(x_ref[...], w_ref[...], preferred_element_type=jnp.float32)
    h = h * dis_ref[...]
    half = h.shape[1] // 2
    out_ref[0] = h[:, :half]
    out_ref[1] = h[:, half:]


def _k2_body(agg_ref, h_ref, dis_ref, b_ref, w_ref, out_ref):
    t = jnp.concatenate([agg_ref[0] + h_ref[0], agg_ref[1] + h_ref[1]], axis=1)
    act = jnp.maximum(t * dis_ref[...] + b_ref[...], 0.0)
    h = jnp.dot(act, w_ref[...], preferred_element_type=jnp.float32)
    h = h * dis_ref[...]
    half = h.shape[1] // 2
    out_ref[0] = h[:, :half]
    out_ref[1] = h[:, half:]


def _k3_body(agg_ref, h_ref, dis_ref, b_ref, w_ref, out_ref):
    t = jnp.concatenate([agg_ref[0] + h_ref[0], agg_ref[1] + h_ref[1]], axis=1)
    act = jnp.maximum(t * dis_ref[...] + b_ref[...], 0.0)
    h = jnp.dot(act, w_ref[...], preferred_element_type=jnp.float32)
    out_ref[...] = h * dis_ref[...]


def _k4_body(agg_ref, h_ref, dis_ref, b_ref, fw1_ref, fb1_ref, fw2_ref,
             fb2_ref, out_ref):
    t = agg_ref[0] + agg_ref[1] + h_ref[...]
    act = jnp.maximum(t * dis_ref[...] + b_ref[...], 0.0)
    hf = jnp.dot(act, fw1_ref[...], preferred_element_type=jnp.float32)
    hf = jnp.maximum(hf + fb1_ref[...], 0.0)
    out = jnp.dot(hf, fw2_ref[...], preferred_element_type=jnp.float32)
    out_ref[...] = out + fb2_ref[...]


def _dis_kernel(deg):
    return pl.pallas_call(
        _k0_body,
        out_shape=jax.ShapeDtypeStruct((N, 1), jnp.float32),
        grid=(GRID,),
        in_specs=[pl.BlockSpec((2, NB, 128), lambda j: (0, j, 0))],
        out_specs=pl.BlockSpec((NB, 1), lambda j: (j, 0)),
    )(deg)


def _h1_kernel(x, w, dis):
    din, dout = w.shape
    return _tc_call(
        _k1_body,
        jax.ShapeDtypeStruct((2, N, dout // 2), jnp.float32),
        [
            pl.BlockSpec((NB, din), lambda j: (j, 0)),
            _full((din, dout)),
            pl.BlockSpec((NB, 1), lambda j: (j, 0)),
        ],
        pl.BlockSpec((2, NB, dout // 2), lambda j: (0, j, 0)),
    )(x, w, dis)


def _mid_kernel(agg, h, dis, b, w):
    din, dout = w.shape
    return _tc_call(
        _k2_body,
        jax.ShapeDtypeStruct((2, N, dout // 2), jnp.float32),
        [
            pl.BlockSpec((2, NB, din // 2), lambda j: (0, j, 0)),
            pl.BlockSpec((2, NB, din // 2), lambda j: (0, j, 0)),
            pl.BlockSpec((NB, 1), lambda j: (j, 0)),
            _full((1, din)),
            _full((din, dout)),
        ],
        pl.BlockSpec((2, NB, dout // 2), lambda j: (0, j, 0)),
    )(agg, h, dis, b, w)


def _mid_full_kernel(agg, h, dis, b, w):
    din, dout = w.shape
    return _tc_call(
        _k3_body,
        jax.ShapeDtypeStruct((N, dout), jnp.float32),
        [
            pl.BlockSpec((2, NB, din // 2), lambda j: (0, j, 0)),
            pl.BlockSpec((2, NB, din // 2), lambda j: (0, j, 0)),
            pl.BlockSpec((NB, 1), lambda j: (j, 0)),
            _full((1, din)),
            _full((din, dout)),
        ],
        pl.BlockSpec((NB, dout), lambda j: (j, 0)),
    )(agg, h, dis, b, w)


def _head_kernel(agg, h, dis, b3, fw1, fb1, fw2, fb2):
    return _tc_call(
        _k4_body,
        jax.ShapeDtypeStruct((N, EMB), jnp.float32),
        [
            pl.BlockSpec((2, NB, EMB), lambda j: (0, j, 0)),
            pl.BlockSpec((NB, EMB), lambda j: (j, 0)),
            pl.BlockSpec((NB, 1), lambda j: (j, 0)),
            _full((1, EMB)),
            _full((EMB, HID)),
            _full((1, HID)),
            _full((HID, EMB)),
            _full((1, EMB)),
        ],
        pl.BlockSpec((NB, EMB), lambda j: (j, 0)),
    )(agg, h, dis, b3, fw1, fb1, fw2, fb2)


def kernel(x, edge_index, W1, b1, W2, b2, W3, b3, fcW1, fcb1, fcW2, fcb2):
    row = edge_index[0]
    col = edge_index[1]

    pad = EP - E
    pad_i = jnp.arange(pad, dtype=row.dtype)
    rowp = jnp.concatenate([row, pad_i % N])
    colp = jnp.concatenate([col, TRASH + pad_i % (ACC_ROWS - N)])
    ridx1 = rowp.reshape(NCHT, CH)
    ridx2 = jnp.stack([rowp, rowp + N]).reshape(NSC, NCHT, CH)
    cidx = colp.reshape(NCHT, CH)
    ones128 = jnp.ones((CH, 128), jnp.float32)
    zeros128 = jnp.zeros((ACC_ROWS, 128), jnp.float32)

    deg = _DEG(cidx, ones128, zeros128)
    dis = _dis_kernel(deg)

    h1 = _h1_kernel(x, W1, dis)
    agg1 = _AGG128(h1.reshape(2 * N, HID // 2), ridx2, cidx, zeros128)
    h2 = _mid_kernel(agg1, h1, dis, b1.reshape(1, HID), W2)
    agg2 = _AGG128(h2.reshape(2 * N, HID // 2), ridx2, cidx, zeros128)
    h3 = _mid_full_kernel(agg2, h2, dis, b2.reshape(1, HID), W3)
    agg3 = _AGGF(h3, ridx1, cidx, zeros128)
    out = _head_kernel(agg3, h3, dis, b3.reshape(1, EMB),
                       fcW1, fcb1.reshape(1, HID), fcW2, fcb2.reshape(1, EMB))
    return out


# speedup vs baseline: 2.6057x; 1.0093x over previous
import functools

import jax
import jax.numpy as jnp
from jax import lax
from jax.experimental import pallas as pl
from jax.experimental.pallas import tpu as pltpu
from jax.experimental.pallas import tpu_sc as plsc

N = 10000
E = 320000
FEAT = 128
HID = 256
EMB = 128

NB = 1000
GRID = N // NB

NSC = 2
NSS = 16
CH = 128
NCH = 160
NCHT = NCH * NSS
EP = NCHT * CH
ACC_ROWS = 10240
ZPT = ACC_ROWS // NSS
SECN = 16
CH2 = 64
NBUF = 4
SROW = 16
CPW = NCHT // (NSC * NSS)
TRASH = N


def _sc_mesh():
    return plsc.VectorSubcoreMesh(core_axis_name="c", subcore_axis_name="s",
                                  num_cores=NSC, num_subcores=NSS)


def _make_agg(dh, split):
    nch = NCH if split else CPW
    secn = SECN if split else SECN // 2
    nsec = nch // secn
    @functools.partial(
        pl.kernel,
        out_type=jax.ShapeDtypeStruct((NSC, ACC_ROWS, dh), jnp.float32),
        mesh=_sc_mesh(),
        scratch_types=[
            [pltpu.VMEM((secn, CH), jnp.int32)] * 2,
            [pltpu.VMEM((secn, CH), jnp.int32)] * 2,
            [pltpu.VMEM((CH, dh), jnp.float32)] * 2,
            [pltpu.SemaphoreType.DMA] * 2,
            [pltpu.SemaphoreType.DMA] * 2,
            [pltpu.SemaphoreType.DMA] * 2,
            pltpu.VMEM_SHARED((ACC_ROWS, dh), jnp.float32),
        ],
    )
    def agg(h, ridx, cidx, zeros, out, rows, cols, bufs, gsem, ssem, isem,
            acc):
        c = lax.axis_index("c")
        s = lax.axis_index("s")
        base = s * nch if split else (c * NSS + s) * CPW

        def ridx_src(t):
            sl = pl.ds(base + t * secn, secn)
            return ridx.at[c, sl] if split else ridx.at[sl]

        def issue_idx(t, par):
            pltpu.async_copy(ridx_src(t), rows[par], isem[par])
            pltpu.async_copy(cidx.at[pl.ds(base + t * secn, secn)], cols[par],
                             isem[par])

        def wait_idx(par):
            pltpu.make_async_copy(ridx_src(0), rows[par], isem[par]).wait()
            pltpu.make_async_copy(ridx_src(0), cols[par], isem[par]).wait()

        def section(t, par):
            wait_idx(par)

            @pl.when(t + 1 < nsec)
            def _():
                issue_idx(t + 1, 1 - par)

            row_v = rows[par]
            col_v = cols[par]
            g = [None, None]
            sc = [None, None]
            g[0] = pltpu.async_copy(h.at[row_v.at[0]], bufs[0], gsem[0])
            for j in range(secn):
                p = j % 2
                q = (j + 1) % 2
                if j + 1 < secn:
                    if j >= 1:
                        sc[q].wait()
                    g[q] = pltpu.async_copy(h.at[row_v.at[j + 1]], bufs[q],
                                            gsem[q])
                g[p].wait()
                sc[p] = pltpu.async_copy(bufs[p], acc.at[col_v.at[j]],
                                         ssem[p], add=True)
            sc[(secn - 2) % 2].wait()
            sc[(secn - 1) % 2].wait()

        issue_idx(0, 0)
        pltpu.sync_copy(zeros.at[pl.ds(s * ZPT, ZPT)],
                        acc.at[pl.ds(s * ZPT, ZPT)])
        plsc.subcore_barrier()

        def pair(u, carry):
            section(2 * u, 0)
            section(2 * u + 1, 1)
            return carry

        lax.fori_loop(0, nsec // 2, pair, 0)
        plsc.subcore_barrier()
        pltpu.sync_copy(acc.at[pl.ds(s * ZPT, ZPT)],
                        out.at[c, pl.ds(s * ZPT, ZPT)])

    return agg


def _make_deg():
    @functools.partial(
        pl.kernel,
        out_type=jax.ShapeDtypeStruct((NSC, ACC_ROWS, 128), jnp.float32),
        mesh=_sc_mesh(),
        scratch_types=[
            pltpu.VMEM((SECN, CH), jnp.int32),
            pltpu.VMEM((CH, 128), jnp.float32),
            pltpu.VMEM_SHARED((ACC_ROWS, 128), jnp.float32),
            pltpu.SemaphoreType.DMA,
        ],
    )
    def deg(cidx, ones, zeros, out, col_v, ones_v, acc, sem):
        c = lax.axis_index("c")
        s = lax.axis_index("s")
        base = (c * NSS + s) * CPW
        pltpu.sync_copy(ones, ones_v)
        pltpu.sync_copy(zeros.at[pl.ds(s * ZPT, ZPT)],
                        acc.at[pl.ds(s * ZPT, ZPT)])
        plsc.subcore_barrier()

        def section(t, carry):
            pltpu.sync_copy(cidx.at[pl.ds(base + t * SECN, SECN)], col_v)
            descs = [pltpu.async_copy(ones_v, acc.at[col_v.at[j]], sem,
                                      add=True) for j in range(SECN)]
            for d in descs:
                d.wait()
            return carry

        lax.fori_loop(0, CPW // SECN, section, 0)
        plsc.subcore_barrier()
        pltpu.sync_copy(acc.at[pl.ds(s * ZPT, ZPT)],
                        out.at[c, pl.ds(s * ZPT, ZPT)])

    return deg


_AGG128 = _make_agg(HID // 2, split=True)
_AGGF = _make_agg(EMB, split=False)
_DEG = _make_deg()


def _tc_call(body, out_shape, in_specs, out_specs):
    return pl.pallas_call(
        body,
        out_shape=out_shape,
        grid=(GRID,),
        in_specs=in_specs,
        out_specs=out_specs,
    )


def _full(shape):
    return pl.BlockSpec(shape, lambda j: tuple(0 for _ in shape))


def _k0_body(deg_ref, h_ref, dis_ref, out_ref):
    dis = lax.rsqrt(deg_ref[0, :, :1] + deg_ref[1, :, :1] + 1.0)
    dis_ref[...] = dis
    out_ref[0] = h_ref[0] * dis
    out_ref[1] = h_ref[1] * dis


def _k1_body(x_ref, w_ref, out_ref):
    h = jnp.dot(x_ref[...], w_ref[...], preferred_element_type=jnp.float32)
    half = h.shape[1] // 2
    out_ref[0] = h[:, :half]
    out_ref[1] = h[:, half:]


def _k2_body(agg_ref, h_ref, dis_ref, b_ref, w_ref, out_ref):
    t = jnp.concatenate([agg_ref[0] + h_ref[0], agg_ref[1] + h_ref[1]], axis=1)
    act = jnp.maximum(t * dis_ref[...] + b_ref[...], 0.0)
    h = jnp.dot(act, w_ref[...], preferred_element_type=jnp.float32)
    h = h * dis_ref[...]
    half = h.shape[1] // 2
    out_ref[0] = h[:, :half]
    out_ref[1] = h[:, half:]


def _k3_body(agg_ref, h_ref, dis_ref, b_ref, w_ref, out_ref):
    t = jnp.concatenate([agg_ref[0] + h_ref[0], agg_ref[1] + h_ref[1]], axis=1)
    act = jnp.maximum(t * dis_ref[...] + b_ref[...], 0.0)
    h = jnp.dot(act, w_ref[...], preferred_element_type=jnp.float32)
    out_ref[...] = h * dis_ref[...]


def _k4_body(agg_ref, h_ref, dis_ref, b_ref, fw1_ref, fb1_ref, fw2_ref,
             fb2_ref, out_ref):
    t = agg_ref[0] + agg_ref[1] + h_ref[...]
    act = jnp.maximum(t * dis_ref[...] + b_ref[...], 0.0)
    hf = jnp.dot(act, fw1_ref[...], preferred_element_type=jnp.float32)
    hf = jnp.maximum(hf + fb1_ref[...], 0.0)
    out = jnp.dot(hf, fw2_ref[...], preferred_element_type=jnp.float32)
    out_ref[...] = out + fb2_ref[...]


def _dis_kernel(deg, h_raw):
    return pl.pallas_call(
        _k0_body,
        out_shape=(jax.ShapeDtypeStruct((N, 1), jnp.float32),
                   jax.ShapeDtypeStruct((2, N, HID // 2), jnp.float32)),
        grid=(GRID,),
        in_specs=[pl.BlockSpec((2, NB, 128), lambda j: (0, j, 0)),
                  pl.BlockSpec((2, NB, HID // 2), lambda j: (0, j, 0))],
        out_specs=(pl.BlockSpec((NB, 1), lambda j: (j, 0)),
                   pl.BlockSpec((2, NB, HID // 2), lambda j: (0, j, 0))),
    )(deg, h_raw)


def _h1_kernel(x, w):
    din, dout = w.shape
    return _tc_call(
        _k1_body,
        jax.ShapeDtypeStruct((2, N, dout // 2), jnp.float32),
        [
            pl.BlockSpec((NB, din), lambda j: (j, 0)),
            _full((din, dout)),
        ],
        pl.BlockSpec((2, NB, dout // 2), lambda j: (0, j, 0)),
    )(x, w)


def _mid_kernel(agg, h, dis, b, w):
    din, dout = w.shape
    return _tc_call(
        _k2_body,
        jax.ShapeDtypeStruct((2, N, dout // 2), jnp.float32),
        [
            pl.BlockSpec((2, NB, din // 2), lambda j: (0, j, 0)),
            pl.BlockSpec((2, NB, din // 2), lambda j: (0, j, 0)),
            pl.BlockSpec((NB, 1), lambda j: (j, 0)),
            _full((1, din)),
            _full((din, dout)),
        ],
        pl.BlockSpec((2, NB, dout // 2), lambda j: (0, j, 0)),
    )(agg, h, dis, b, w)


def _mid_full_kernel(agg, h, dis, b, w):
    din, dout = w.shape
    return _tc_call(
        _k3_body,
        jax.ShapeDtypeStruct((N, dout), jnp.float32),
        [
            pl.BlockSpec((2, NB, din // 2), lambda j: (0, j, 0)),
            pl.BlockSpec((2, NB, din // 2), lambda j: (0, j, 0)),
            pl.BlockSpec((NB, 1), lambda j: (j, 0)),
            _full((1, din)),
            _full((din, dout)),
        ],
        pl.BlockSpec((NB, dout), lambda j: (j, 0)),
    )(agg, h, dis, b, w)


def _head_kernel(agg, h, dis, b3, fw1, fb1, fw2, fb2):
    return _tc_call(
        _k4_body,
        jax.ShapeDtypeStruct((N, EMB), jnp.float32),
        [
            pl.BlockSpec((2, NB, EMB), lambda j: (0, j, 0)),
            pl.BlockSpec((NB, EMB), lambda j: (j, 0)),
            pl.BlockSpec((NB, 1), lambda j: (j, 0)),
            _full((1, EMB)),
            _full((EMB, HID)),
            _full((1, HID)),
            _full((HID, EMB)),
            _full((1, EMB)),
        ],
        pl.BlockSpec((NB, EMB), lambda j: (j, 0)),
    )(agg, h, dis, b3, fw1, fb1, fw2, fb2)


def kernel(x, edge_index, W1, b1, W2, b2, W3, b3, fcW1, fcb1, fcW2, fcb2):
    row = edge_index[0]
    col = edge_index[1]

    pad = EP - E
    pad_i = jnp.arange(pad, dtype=row.dtype)
    rowp = jnp.concatenate([row, pad_i % N])
    colp = jnp.concatenate([col, TRASH + pad_i % (ACC_ROWS - N)])
    ridx1 = rowp.reshape(NCHT, CH)
    ridx2 = jnp.stack([rowp, rowp + N]).reshape(NSC, NCHT, CH)
    cidx = colp.reshape(NCHT, CH)
    ones128 = jnp.ones((CH, 128), jnp.float32)
    zeros128 = jnp.zeros((ACC_ROWS, 128), jnp.float32)

    h1_raw = _h1_kernel(x, W1)
    deg = _DEG(cidx, ones128, zeros128)
    dis, h1 = _dis_kernel(deg, h1_raw)

    agg1 = _AGG128(h1.reshape(2 * N, HID // 2), ridx2, cidx, zeros128)
    h2 = _mid_kernel(agg1, h1, dis, b1.reshape(1, HID), W2)
    agg2 = _AGG128(h2.reshape(2 * N, HID // 2), ridx2, cidx, zeros128)
    h3 = _mid_full_kernel(agg2, h2, dis, b2.reshape(1, HID), W3)
    agg3 = _AGGF(h3, ridx1, cidx, zeros128)
    out = _head_kernel(agg3, h3, dis, b3.reshape(1, EMB),
                       fcW1, fcb1.reshape(1, HID), fcW2, fcb2.reshape(1, EMB))
    return out


# speedup vs baseline: 2.6272x; 1.0082x over previous
import functools

import jax
import jax.numpy as jnp
from jax import lax
from jax.experimental import pallas as pl
from jax.experimental.pallas import tpu as pltpu
from jax.experimental.pallas import tpu_sc as plsc

N = 10000
E = 320000
FEAT = 128
HID = 256
EMB = 128

NB = 2000
GRID = N // NB

NSC = 2
NSS = 16
CH = 128
NCH = 160
NCHT = NCH * NSS
EP = NCHT * CH
ACC_ROWS = 10240
ZPT = ACC_ROWS // NSS
SECN = 16
CH2 = 64
NBUF = 4
SROW = 16
CPW = NCHT // (NSC * NSS)
TRASH = N


def _sc_mesh():
    return plsc.VectorSubcoreMesh(core_axis_name="c", subcore_axis_name="s",
                                  num_cores=NSC, num_subcores=NSS)


def _make_agg(dh, split):
    nch = NCH if split else CPW
    secn = SECN if split else SECN // 2
    nsec = nch // secn
    @functools.partial(
        pl.kernel,
        out_type=jax.ShapeDtypeStruct((NSC, ACC_ROWS, dh), jnp.float32),
        mesh=_sc_mesh(),
        scratch_types=[
            [pltpu.VMEM((secn, CH), jnp.int32)] * 2,
            [pltpu.VMEM((secn, CH), jnp.int32)] * 2,
            [pltpu.VMEM((CH, dh), jnp.float32)] * 2,
            [pltpu.SemaphoreType.DMA] * 2,
            [pltpu.SemaphoreType.DMA] * 2,
            [pltpu.SemaphoreType.DMA] * 2,
            pltpu.VMEM_SHARED((ACC_ROWS, dh), jnp.float32),
        ],
    )
    def agg(h, ridx, cidx, zeros, out, rows, cols, bufs, gsem, ssem, isem,
            acc):
        c = lax.axis_index("c")
        s = lax.axis_index("s")
        base = s * nch if split else (c * NSS + s) * CPW

        def ridx_src(t):
            sl = pl.ds(base + t * secn, secn)
            return ridx.at[c, sl] if split else ridx.at[sl]

        def issue_idx(t, par):
            pltpu.async_copy(ridx_src(t), rows[par], isem[par])
            pltpu.async_copy(cidx.at[pl.ds(base + t * secn, secn)], cols[par],
                             isem[par])

        def wait_idx(par):
            pltpu.make_async_copy(ridx_src(0), rows[par], isem[par]).wait()
            pltpu.make_async_copy(ridx_src(0), cols[par], isem[par]).wait()

        def section(t, par):
            wait_idx(par)

            @pl.when(t + 1 < nsec)
            def _():
                issue_idx(t + 1, 1 - par)

            row_v = rows[par]
            col_v = cols[par]
            g = [None, None]
            sc = [None, None]
            g[0] = pltpu.async_copy(h.at[row_v.at[0]], bufs[0], gsem[0])
            for j in range(secn):
                p = j % 2
                q = (j + 1) % 2
                if j + 1 < secn:
                    if j >= 1:
                        sc[q].wait()
                    g[q] = pltpu.async_copy(h.at[row_v.at[j + 1]], bufs[q],
                                            gsem[q])
                g[p].wait()
                sc[p] = pltpu.async_copy(bufs[p], acc.at[col_v.at[j]],
                                         ssem[p], add=True)
            sc[(secn - 2) % 2].wait()
            sc[(secn - 1) % 2].wait()

        issue_idx(0, 0)
        pltpu.sync_copy(zeros.at[pl.ds(s * ZPT, ZPT)],
                        acc.at[pl.ds(s * ZPT, ZPT)])
        plsc.subcore_barrier()

        def pair(u, carry):
            section(2 * u, 0)
            section(2 * u + 1, 1)
            return carry

        lax.fori_loop(0, nsec // 2, pair, 0)
        plsc.subcore_barrier()
        pltpu.sync_copy(acc.at[pl.ds(s * ZPT, ZPT)],
                        out.at[c, pl.ds(s * ZPT, ZPT)])

    return agg


def _make_deg():
    @functools.partial(
        pl.kernel,
        out_type=jax.ShapeDtypeStruct((NSC, ACC_ROWS, 128), jnp.float32),
        mesh=_sc_mesh(),
        scratch_types=[
            pltpu.VMEM((SECN, CH), jnp.int32),
            pltpu.VMEM((CH, 128), jnp.float32),
            pltpu.VMEM_SHARED((ACC_ROWS, 128), jnp.float32),
            pltpu.SemaphoreType.DMA,
        ],
    )
    def deg(cidx, ones, zeros, out, col_v, ones_v, acc, sem):
        c = lax.axis_index("c")
        s = lax.axis_index("s")
        base = (c * NSS + s) * CPW
        pltpu.sync_copy(ones, ones_v)
        pltpu.sync_copy(zeros.at[pl.ds(s * ZPT, ZPT)],
                        acc.at[pl.ds(s * ZPT, ZPT)])
        plsc.subcore_barrier()

        def section(t, carry):
            pltpu.sync_copy(cidx.at[pl.ds(base + t * SECN, SECN)], col_v)
            descs = [pltpu.async_copy(ones_v, acc.at[col_v.at[j]], sem,
                                      add=True) for j in range(SECN)]
            for d in descs:
                d.wait()
            return carry

        lax.fori_loop(0, CPW // SECN, section, 0)
        plsc.subcore_barrier()
        pltpu.sync_copy(acc.at[pl.ds(s * ZPT, ZPT)],
                        out.at[c, pl.ds(s * ZPT, ZPT)])

    return deg


_AGG128 = _make_agg(HID // 2, split=True)
_AGGF = _make_agg(EMB, split=False)
_DEG = _make_deg()


def _tc_call(body, out_shape, in_specs, out_specs):
    return pl.pallas_call(
        body,
        out_shape=out_shape,
        grid=(GRID,),
        in_specs=in_specs,
        out_specs=out_specs,
    )


def _full(shape):
    return pl.BlockSpec(shape, lambda j: tuple(0 for _ in shape))


def _k0_body(deg_ref, h_ref, dis_ref, out_ref):
    dis = lax.rsqrt(deg_ref[0, :, :1] + deg_ref[1, :, :1] + 1.0)
    dis_ref[...] = dis
    out_ref[0] = h_ref[0] * dis
    out_ref[1] = h_ref[1] * dis


def _k1_body(x_ref, w_ref, out_ref):
    h = jnp.dot(x_ref[...], w_ref[...], preferred_element_type=jnp.float32)
    half = h.shape[1] // 2
    out_ref[0] = h[:, :half]
    out_ref[1] = h[:, half:]


def _k2_body(agg_ref, h_ref, dis_ref, b_ref, w_ref, out_ref):
    t = jnp.concatenate([agg_ref[0] + h_ref[0], agg_ref[1] + h_ref[1]], axis=1)
    act = jnp.maximum(t * dis_ref[...] + b_ref[...], 0.0)
    h = jnp.dot(act, w_ref[...], preferred_element_type=jnp.float32)
    h = h * dis_ref[...]
    half = h.shape[1] // 2
    out_ref[0] = h[:, :half]
    out_ref[1] = h[:, half:]


def _k3_body(agg_ref, h_ref, dis_ref, b_ref, w_ref, out_ref):
    t = jnp.concatenate([agg_ref[0] + h_ref[0], agg_ref[1] + h_ref[1]], axis=1)
    act = jnp.maximum(t * dis_ref[...] + b_ref[...], 0.0)
    h = jnp.dot(act, w_ref[...], preferred_element_type=jnp.float32)
    out_ref[...] = h * dis_ref[...]


def _k4_body(agg_ref, h_ref, dis_ref, b_ref, fw1_ref, fb1_ref, fw2_ref,
             fb2_ref, out_ref):
    t = agg_ref[0] + agg_ref[1] + h_ref[...]
    act = jnp.maximum(t * dis_ref[...] + b_ref[...], 0.0)
    hf = jnp.dot(act, fw1_ref[...], preferred_element_type=jnp.float32)
    hf = jnp.maximum(hf + fb1_ref[...], 0.0)
    out = jnp.dot(hf, fw2_ref[...], preferred_element_type=jnp.float32)
    out_ref[...] = out + fb2_ref[...]


def _dis_kernel(deg, h_raw):
    return pl.pallas_call(
        _k0_body,
        out_shape=(jax.ShapeDtypeStruct((N, 1), jnp.float32),
                   jax.ShapeDtypeStruct((2, N, HID // 2), jnp.float32)),
        grid=(GRID,),
        in_specs=[pl.BlockSpec((2, NB, 128), lambda j: (0, j, 0)),
                  pl.BlockSpec((2, NB, HID // 2), lambda j: (0, j, 0))],
        out_specs=(pl.BlockSpec((NB, 1), lambda j: (j, 0)),
                   pl.BlockSpec((2, NB, HID // 2), lambda j: (0, j, 0))),
    )(deg, h_raw)


def _h1_kernel(x, w):
    din, dout = w.shape
    return _tc_call(
        _k1_body,
        jax.ShapeDtypeStruct((2, N, dout // 2), jnp.float32),
        [
            pl.BlockSpec((NB, din), lambda j: (j, 0)),
            _full((din, dout)),
        ],
        pl.BlockSpec((2, NB, dout // 2), lambda j: (0, j, 0)),
    )(x, w)


def _mid_kernel(agg, h, dis, b, w):
    din, dout = w.shape
    return _tc_call(
        _k2_body,
        jax.ShapeDtypeStruct((2, N, dout // 2), jnp.float32),
        [
            pl.BlockSpec((2, NB, din // 2), lambda j: (0, j, 0)),
            pl.BlockSpec((2, NB, din // 2), lambda j: (0, j, 0)),
            pl.BlockSpec((NB, 1), lambda j: (j, 0)),
            _full((1, din)),
            _full((din, dout)),
        ],
        pl.BlockSpec((2, NB, dout // 2), lambda j: (0, j, 0)),
    )(agg, h, dis, b, w)


def _mid_full_kernel(agg, h, dis, b, w):
    din, dout = w.shape
    return _tc_call(
        _k3_body,
        jax.ShapeDtypeStruct((N, dout), jnp.float32),
        [
            pl.BlockSpec((2, NB, din // 2), lambda j: (0, j, 0)),
            pl.BlockSpec((2, NB, din // 2), lambda j: (0, j, 0)),
            pl.BlockSpec((NB, 1), lambda j: (j, 0)),
            _full((1, din)),
            _full((din, dout)),
        ],
        pl.BlockSpec((NB, dout), lambda j: (j, 0)),
    )(agg, h, dis, b, w)


def _head_kernel(agg, h, dis, b3, fw1, fb1, fw2, fb2):
    return _tc_call(
        _k4_body,
        jax.ShapeDtypeStruct((N, EMB), jnp.float32),
        [
            pl.BlockSpec((2, NB, EMB), lambda j: (0, j, 0)),
            pl.BlockSpec((NB, EMB), lambda j: (j, 0)),
            pl.BlockSpec((NB, 1), lambda j: (j, 0)),
            _full((1, EMB)),
            _full((EMB, HID)),
            _full((1, HID)),
            _full((HID, EMB)),
            _full((1, EMB)),
        ],
        pl.BlockSpec((NB, EMB), lambda j: (j, 0)),
    )(agg, h, dis, b3, fw1, fb1, fw2, fb2)


def kernel(x, edge_index, W1, b1, W2, b2, W3, b3, fcW1, fcb1, fcW2, fcb2):
    row = edge_index[0]
    col = edge_index[1]

    pad = EP - E
    pad_i = jnp.arange(pad, dtype=row.dtype)
    rowp = jnp.concatenate([row, pad_i % N])
    colp = jnp.concatenate([col, TRASH + pad_i % (ACC_ROWS - N)])
    ridx1 = rowp.reshape(NCHT, CH)
    ridx2 = jnp.stack([rowp, rowp + N]).reshape(NSC, NCHT, CH)
    cidx = colp.reshape(NCHT, CH)
    ones128 = jnp.ones((CH, 128), jnp.float32)
    zeros128 = jnp.zeros((ACC_ROWS, 128), jnp.float32)

    h1_raw = _h1_kernel(x, W1)
    deg = _DEG(cidx, ones128, zeros128)
    dis, h1 = _dis_kernel(deg, h1_raw)

    agg1 = _AGG128(h1.reshape(2 * N, HID // 2), ridx2, cidx, zeros128)
    h2 = _mid_kernel(agg1, h1, dis, b1.reshape(1, HID), W2)
    agg2 = _AGG128(h2.reshape(2 * N, HID // 2), ridx2, cidx, zeros128)
    h3 = _mid_full_kernel(agg2, h2, dis, b2.reshape(1, HID), W3)
    agg3 = _AGGF(h3, ridx1, cidx, zeros128)
    out = _head_kernel(agg3, h3, dis, b3.reshape(1, EMB),
                       fcW1, fcb1.reshape(1, HID), fcW2, fcb2.reshape(1, EMB))
    return out
